# Initial kernel scaffold; baseline (speedup 1.0000x reference)
import jax
import jax.numpy as jnp
from jax.experimental import pallas as pl


def kernel(x, edge_index, batch, W1, b1, W2, b2, W3, b3, Wg, bg):
    raise NotImplementedError("write your pallas kernel here")


# speedup vs baseline: 1.0094x; 1.0094x over previous
import functools

import jax
import jax.numpy as jnp
from jax import lax
from jax.experimental import pallas as pl
from jax.experimental.pallas import tpu as pltpu
from jax.experimental.pallas import tpu_sc as plsc

N_NODES = 10000
N_EDGES = 160000
D = 256
H = 512
NUM_GRAPHS = 64

_NC, _NS, _NL = 2, 16, 16
_NW = _NC * _NS

CHUNK = 128
E_PAD = 163840
_PER_W = E_PAD // _NW
_CHUNKS_PER_W = _PER_W // CHUNK

E_BLK = 1280
N_BLK = 80
GRID = N_EDGES // E_BLK


def _sc_gather_body(x_hbm, src_hbm, dst_hbm, batch_hbm,
                    gs_hbm, gd_hbm, ge_hbm,
                    batch_v, si, di, gsb, gdb, geb, sem):
    wid = lax.axis_index("s") * _NC + lax.axis_index("c")
    pltpu.sync_copy(batch_hbm, batch_v)
    span = wid * _PER_W

    def chunk(k, carry):
        base = pl.multiple_of(span + k * CHUNK, CHUNK)
        pltpu.sync_copy(src_hbm.at[pl.ds(base, CHUNK)], si)
        pltpu.sync_copy(dst_hbm.at[pl.ds(base, CHUNK)], di)
        pltpu.async_copy(x_hbm.at[si], gsb, sem).wait()
        pltpu.async_copy(x_hbm.at[di], gdb, sem).wait()
        for j in range(CHUNK // _NL):
            idx = si[pl.ds(j * _NL, _NL)]
            geb[pl.ds(j * _NL, _NL)] = plsc.load_gather(batch_v, [idx])
        pltpu.sync_copy(gsb, gs_hbm.at[pl.ds(base, CHUNK)])
        pltpu.sync_copy(gdb, gd_hbm.at[pl.ds(base, CHUNK)])
        pltpu.sync_copy(geb, ge_hbm.at[pl.ds(base, CHUNK)])
        return carry

    lax.fori_loop(0, _CHUNKS_PER_W, chunk, 0)


@functools.cache
def _sc_gather_kernel():
    return functools.partial(
        pl.kernel,
        mesh=plsc.VectorSubcoreMesh(core_axis_name="c", subcore_axis_name="s"),
        compiler_params=pltpu.CompilerParams(needs_layout_passes=False),
        out_type=[
            jax.ShapeDtypeStruct((E_PAD, D), jnp.float32),
            jax.ShapeDtypeStruct((E_PAD, D), jnp.float32),
            jax.ShapeDtypeStruct((E_PAD,), jnp.int32),
        ],
        scratch_types=[
            pltpu.VMEM((N_NODES,), jnp.int32),
            pltpu.VMEM((CHUNK,), jnp.int32),
            pltpu.VMEM((CHUNK,), jnp.int32),
            pltpu.VMEM((CHUNK, D), jnp.float32),
            pltpu.VMEM((CHUNK, D), jnp.float32),
            pltpu.VMEM((CHUNK,), jnp.int32),
            pltpu.SemaphoreType.DMA,
        ],
    )(_sc_gather_body)


def _tc_body(gs_ref, gd_ref, ge_ref, x_ref, b_ref,
             w1a_ref, w1b_ref, w2_ref, w3_ref,
             b1_ref, b2_ref, b3_ref, wg_ref, acc_ref):
    i = pl.program_id(0)

    @pl.when(i == 0)
    def _init():
        acc_ref[...] = jnp.zeros_like(acc_ref)

    f32 = jnp.float32
    hi = jax.lax.Precision.HIGHEST
    h1 = jnp.maximum(
        jnp.dot(gs_ref[...], w1a_ref[...], preferred_element_type=f32,
                precision=hi)
        + jnp.dot(gd_ref[...], w1b_ref[...], preferred_element_type=f32,
                  precision=hi)
        + b1_ref[...], 0.0)
    h2 = jnp.maximum(
        jnp.dot(h1, w2_ref[...], preferred_element_type=f32, precision=hi)
        + b2_ref[...], 0.0)

    wg = wg_ref[...]
    wgb = wg[:, D:]
    vrow = jnp.sum(w3_ref[...] * wgb, axis=1)[None, :]
    c = jnp.sum(b3_ref[...] * wgb)
    s_col = jnp.sum(h2 * vrow, axis=1, keepdims=True) + c

    ge = ge_ref[0]
    iota_e = lax.broadcasted_iota(jnp.int32, (E_BLK, 128), 1)
    mask_e = iota_e == ge
    edge_part = jnp.sum(jnp.where(mask_e, s_col, 0.0), axis=0)

    t_col = jnp.sum(x_ref[...] * wg[:, :D], axis=1, keepdims=True)
    bt = b_ref[0]
    iota_n = lax.broadcasted_iota(jnp.int32, (N_BLK, 128), 1)
    mask_n = iota_n == bt
    node_part = jnp.sum(jnp.where(mask_n, t_col, 0.0), axis=0)
    cnt_part = jnp.sum(mask_n.astype(f32), axis=0)

    acc_ref[0:1, :] = acc_ref[0:1, :] + edge_part[None, :]
    acc_ref[1:2, :] = acc_ref[1:2, :] + node_part[None, :]
    acc_ref[2:3, :] = acc_ref[2:3, :] + cnt_part[None, :]


_tc_mlp = pl.pallas_call(
    _tc_body,
    grid=(GRID,),
    in_specs=[
        pl.BlockSpec((E_BLK, D), lambda i: (i, 0)),
        pl.BlockSpec((E_BLK, D), lambda i: (i, 0)),
        pl.BlockSpec((1, E_BLK, 1), lambda i: (i, 0, 0)),
        pl.BlockSpec((N_BLK, D), lambda i: (i, 0)),
        pl.BlockSpec((1, N_BLK, 1), lambda i: (i, 0, 0)),
        pl.BlockSpec((D, H), lambda i: (0, 0)),
        pl.BlockSpec((D, H), lambda i: (0, 0)),
        pl.BlockSpec((H, H), lambda i: (0, 0)),
        pl.BlockSpec((H, D), lambda i: (0, 0)),
        pl.BlockSpec((1, H), lambda i: (0, 0)),
        pl.BlockSpec((1, H), lambda i: (0, 0)),
        pl.BlockSpec((1, D), lambda i: (0, 0)),
        pl.BlockSpec((1, 2 * D), lambda i: (0, 0)),
    ],
    out_specs=pl.BlockSpec((8, 128), lambda i: (0, 0)),
    out_shape=jax.ShapeDtypeStruct((8, 128), jnp.float32),
)


def kernel(x, edge_index, batch, W1, b1, W2, b2, W3, b3, Wg, bg):
    src = edge_index[0].astype(jnp.int32)
    dst = edge_index[1].astype(jnp.int32)
    srcp = jnp.pad(src, (0, E_PAD - N_EDGES))
    dstp = jnp.pad(dst, (0, E_PAD - N_EDGES))
    batch32 = batch.astype(jnp.int32)

    gs, gd, ge = _sc_gather_kernel()(x, srcp, dstp, batch32)

    ge3 = ge[:N_EDGES].reshape(GRID, E_BLK, 1)
    batch3 = batch32.reshape(GRID, N_BLK, 1)

    acc = _tc_mlp(gs, gd, ge3, x, batch3,
                  W1[:D], W1[D:], W2, W3,
                  b1.reshape(1, H), b2.reshape(1, H), b3.reshape(1, D),
                  Wg.reshape(1, 2 * D))

    cnt = acc[2, :NUM_GRAPHS]
    sums = acc[0, :NUM_GRAPHS] + acc[1, :NUM_GRAPHS] + cnt * bg[0]
    return (sums / jnp.maximum(cnt, 1.0))[:, None]


# speedup vs baseline: 1.0807x; 1.0706x over previous
import functools

import jax
import jax.numpy as jnp
from jax import lax
from jax.experimental import pallas as pl
from jax.experimental.pallas import tpu as pltpu
from jax.experimental.pallas import tpu_sc as plsc

N_NODES = 10000
N_EDGES = 160000
D = 256
H = 512
NUM_GRAPHS = 64

_NC, _NS, _NL = 2, 16, 16
_NW = _NC * _NS

CHUNK = 32
NBUF = 4
E_PAD = 163840
_PER_W = E_PAD // _NW
_CHUNKS_PER_W = _PER_W // CHUNK

E_BLK = 1280
N_BLK = 80
GRID = N_EDGES // E_BLK


def _sc_gather_body(x_hbm, src_hbm, dst_hbm, batch_hbm,
                    gs_hbm, gd_hbm, ge_hbm,
                    batch_v, si_all, di_all,
                    gsb0, gsb1, gsb2, gsb3, gdb0, gdb1, gdb2, gdb3,
                    geb0, geb1, geb2, geb3,
                    sg0, sg1, sg2, sg3, sw0, sw1, sw2, sw3):
    gsb = (gsb0, gsb1, gsb2, gsb3)
    gdb = (gdb0, gdb1, gdb2, gdb3)
    geb = (geb0, geb1, geb2, geb3)
    sg = (sg0, sg1, sg2, sg3)
    sw = (sw0, sw1, sw2, sw3)

    wid = lax.axis_index("s") * _NC + lax.axis_index("c")
    span = wid * _PER_W
    pltpu.sync_copy(batch_hbm, batch_v)
    pltpu.sync_copy(src_hbm.at[pl.ds(span, _PER_W)], si_all)
    pltpu.sync_copy(dst_hbm.at[pl.ds(span, _PER_W)], di_all)

    def start(k, b):
        off = pl.multiple_of(k * CHUNK, CHUNK)
        pltpu.async_copy(x_hbm.at[si_all.at[pl.ds(off, CHUNK)]], gsb[b], sg[b])
        pltpu.async_copy(x_hbm.at[di_all.at[pl.ds(off, CHUNK)]], gdb[b], sg[b])

    def wait_gathers(k, b):
        off = pl.multiple_of(k * CHUNK, CHUNK)
        pltpu.make_async_copy(
            x_hbm.at[si_all.at[pl.ds(off, CHUNK)]], gsb[b], sg[b]).wait()
        pltpu.make_async_copy(
            x_hbm.at[di_all.at[pl.ds(off, CHUNK)]], gdb[b], sg[b]).wait()

    def write(k, b):
        base = pl.multiple_of(span + k * CHUNK, CHUNK)
        pltpu.async_copy(gsb[b], gs_hbm.at[pl.ds(base, CHUNK)], sw[b])
        pltpu.async_copy(gdb[b], gd_hbm.at[pl.ds(base, CHUNK)], sw[b])
        pltpu.async_copy(geb[b], ge_hbm.at[pl.ds(base, CHUNK)], sw[b])

    def wait_writes(b):
        pltpu.make_async_copy(gsb[b], gs_hbm.at[pl.ds(span, CHUNK)], sw[b]).wait()
        pltpu.make_async_copy(gdb[b], gd_hbm.at[pl.ds(span, CHUNK)], sw[b]).wait()
        pltpu.make_async_copy(geb[b], ge_hbm.at[pl.ds(span, CHUNK)], sw[b]).wait()

    for b in range(NBUF - 1):
        start(b, b)

    def body(p, carry):
        for b in range(NBUF):
            k = p * NBUF + b
            kn = k + NBUF - 1
            bn = (b + NBUF - 1) % NBUF

            @pl.when(kn < _CHUNKS_PER_W)
            def _():
                @pl.when(kn >= NBUF)
                def _():
                    wait_writes(bn)
                start(kn, bn)

            wait_gathers(k, b)
            for j in range(CHUNK // _NL):
                idx = si_all[pl.ds(k * CHUNK + j * _NL, _NL)]
                geb[b][pl.ds(j * _NL, _NL)] = plsc.load_gather(batch_v, [idx])
            write(k, b)
        return carry

    lax.fori_loop(0, _CHUNKS_PER_W // NBUF, body, 0)
    for b in range(NBUF):
        wait_writes(b)


@functools.cache
def _sc_gather_kernel():
    return functools.partial(
        pl.kernel,
        mesh=plsc.VectorSubcoreMesh(core_axis_name="c", subcore_axis_name="s"),
        compiler_params=pltpu.CompilerParams(needs_layout_passes=False),
        out_type=[
            jax.ShapeDtypeStruct((E_PAD, D), jnp.float32),
            jax.ShapeDtypeStruct((E_PAD, D), jnp.float32),
            jax.ShapeDtypeStruct((E_PAD,), jnp.int32),
        ],
        scratch_types=(
            [pltpu.VMEM((N_NODES,), jnp.int32),
             pltpu.VMEM((_PER_W,), jnp.int32),
             pltpu.VMEM((_PER_W,), jnp.int32)]
            + [pltpu.VMEM((CHUNK, D), jnp.float32)] * (2 * NBUF)
            + [pltpu.VMEM((CHUNK,), jnp.int32)] * NBUF
            + [pltpu.SemaphoreType.DMA] * (2 * NBUF)
        ),
    )(_sc_gather_body)


def _tc_body(gs_ref, gd_ref, ge_ref, x_ref, b_ref,
             w1a_ref, w1b_ref, w2_ref, w3_ref,
             b1_ref, b2_ref, b3_ref, wg_ref, acc_ref):
    i = pl.program_id(0)

    @pl.when(i == 0)
    def _init():
        acc_ref[...] = jnp.zeros_like(acc_ref)

    f32 = jnp.float32
    hi = jax.lax.Precision.HIGHEST
    h1 = jnp.maximum(
        jnp.dot(gs_ref[...], w1a_ref[...], preferred_element_type=f32,
                precision=hi)
        + jnp.dot(gd_ref[...], w1b_ref[...], preferred_element_type=f32,
                  precision=hi)
        + b1_ref[...], 0.0)
    h2 = jnp.maximum(
        jnp.dot(h1, w2_ref[...], preferred_element_type=f32, precision=hi)
        + b2_ref[...], 0.0)

    wg = wg_ref[...]
    wgb = wg[:, D:]
    vrow = jnp.sum(w3_ref[...] * wgb, axis=1)[None, :]
    c = jnp.sum(b3_ref[...] * wgb)
    s_col = jnp.sum(h2 * vrow, axis=1, keepdims=True) + c

    ge = ge_ref[0]
    iota_e = lax.broadcasted_iota(jnp.int32, (E_BLK, 128), 1)
    mask_e = iota_e == ge
    edge_part = jnp.sum(jnp.where(mask_e, s_col, 0.0), axis=0)

    t_col = jnp.sum(x_ref[...] * wg[:, :D], axis=1, keepdims=True)
    bt = b_ref[0]
    iota_n = lax.broadcasted_iota(jnp.int32, (N_BLK, 128), 1)
    mask_n = iota_n == bt
    node_part = jnp.sum(jnp.where(mask_n, t_col, 0.0), axis=0)
    cnt_part = jnp.sum(mask_n.astype(f32), axis=0)

    acc_ref[0:1, :] = acc_ref[0:1, :] + edge_part[None, :]
    acc_ref[1:2, :] = acc_ref[1:2, :] + node_part[None, :]
    acc_ref[2:3, :] = acc_ref[2:3, :] + cnt_part[None, :]


_tc_mlp = pl.pallas_call(
    _tc_body,
    grid=(GRID,),
    in_specs=[
        pl.BlockSpec((E_BLK, D), lambda i: (i, 0)),
        pl.BlockSpec((E_BLK, D), lambda i: (i, 0)),
        pl.BlockSpec((1, E_BLK, 1), lambda i: (i, 0, 0)),
        pl.BlockSpec((N_BLK, D), lambda i: (i, 0)),
        pl.BlockSpec((1, N_BLK, 1), lambda i: (i, 0, 0)),
        pl.BlockSpec((D, H), lambda i: (0, 0)),
        pl.BlockSpec((D, H), lambda i: (0, 0)),
        pl.BlockSpec((H, H), lambda i: (0, 0)),
        pl.BlockSpec((H, D), lambda i: (0, 0)),
        pl.BlockSpec((1, H), lambda i: (0, 0)),
        pl.BlockSpec((1, H), lambda i: (0, 0)),
        pl.BlockSpec((1, D), lambda i: (0, 0)),
        pl.BlockSpec((1, 2 * D), lambda i: (0, 0)),
    ],
    out_specs=pl.BlockSpec((8, 128), lambda i: (0, 0)),
    out_shape=jax.ShapeDtypeStruct((8, 128), jnp.float32),
)


def kernel(x, edge_index, batch, W1, b1, W2, b2, W3, b3, Wg, bg):
    src = edge_index[0].astype(jnp.int32)
    dst = edge_index[1].astype(jnp.int32)
    srcp = jnp.pad(src, (0, E_PAD - N_EDGES))
    dstp = jnp.pad(dst, (0, E_PAD - N_EDGES))
    batch32 = batch.astype(jnp.int32)

    gs, gd, ge = _sc_gather_kernel()(x, srcp, dstp, batch32)

    ge3 = ge[:N_EDGES].reshape(GRID, E_BLK, 1)
    batch3 = batch32.reshape(GRID, N_BLK, 1)

    acc = _tc_mlp(gs, gd, ge3, x, batch3,
                  W1[:D], W1[D:], W2, W3,
                  b1.reshape(1, H), b2.reshape(1, H), b3.reshape(1, D),
                  Wg.reshape(1, 2 * D))

    cnt = acc[2, :NUM_GRAPHS]
    sums = acc[0, :NUM_GRAPHS] + acc[1, :NUM_GRAPHS] + cnt * bg[0]
    return (sums / jnp.maximum(cnt, 1.0))[:, None]


# speedup vs baseline: 1.1575x; 1.0710x over previous
import functools

import jax
import jax.numpy as jnp
from jax import lax
from jax.experimental import pallas as pl
from jax.experimental.pallas import tpu as pltpu
from jax.experimental.pallas import tpu_sc as plsc

N_NODES = 10000
N_EDGES = 160000
D = 256
H = 512
NUM_GRAPHS = 64

_NC, _NS, _NL = 2, 16, 16
_NW = _NC * _NS

CHUNK = 32
NBUF = 4
E_PAD = 163840
E_HALF = E_PAD // 2

E_BLK = 1280
N_BLK = 1000


def _make_sc_gather_body(per_w):
    chunks_per_w = per_w // CHUNK

    def body(x_hbm, src_hbm, dst_hbm, batch_hbm,
             gs_hbm, gd_hbm, ge_hbm,
             batch_v, si_all, di_all,
             gsb0, gsb1, gsb2, gsb3, gdb0, gdb1, gdb2, gdb3,
             geb0, geb1, geb2, geb3,
             sg0, sg1, sg2, sg3, sw0, sw1, sw2, sw3):
        gsb = (gsb0, gsb1, gsb2, gsb3)
        gdb = (gdb0, gdb1, gdb2, gdb3)
        geb = (geb0, geb1, geb2, geb3)
        sg = (sg0, sg1, sg2, sg3)
        sw = (sw0, sw1, sw2, sw3)

        wid = lax.axis_index("s") * _NC + lax.axis_index("c")
        span = wid * per_w
        pltpu.sync_copy(batch_hbm, batch_v)
        pltpu.sync_copy(src_hbm.at[pl.ds(span, per_w)], si_all)
        pltpu.sync_copy(dst_hbm.at[pl.ds(span, per_w)], di_all)

        def start(k, b):
            off = pl.multiple_of(k * CHUNK, CHUNK)
            pltpu.async_copy(x_hbm.at[si_all.at[pl.ds(off, CHUNK)]],
                             gsb[b], sg[b])
            pltpu.async_copy(x_hbm.at[di_all.at[pl.ds(off, CHUNK)]],
                             gdb[b], sg[b])

        def wait_gathers(k, b):
            off = pl.multiple_of(k * CHUNK, CHUNK)
            pltpu.make_async_copy(
                x_hbm.at[si_all.at[pl.ds(off, CHUNK)]], gsb[b], sg[b]).wait()
            pltpu.make_async_copy(
                x_hbm.at[di_all.at[pl.ds(off, CHUNK)]], gdb[b], sg[b]).wait()

        def write(k, b):
            base = pl.multiple_of(span + k * CHUNK, CHUNK)
            pltpu.async_copy(gsb[b], gs_hbm.at[pl.ds(base, CHUNK)], sw[b])
            pltpu.async_copy(gdb[b], gd_hbm.at[pl.ds(base, CHUNK)], sw[b])
            pltpu.async_copy(geb[b], ge_hbm.at[pl.ds(base, CHUNK)], sw[b])

        def wait_writes(b):
            pltpu.make_async_copy(
                gsb[b], gs_hbm.at[pl.ds(span, CHUNK)], sw[b]).wait()
            pltpu.make_async_copy(
                gdb[b], gd_hbm.at[pl.ds(span, CHUNK)], sw[b]).wait()
            pltpu.make_async_copy(
                geb[b], ge_hbm.at[pl.ds(span, CHUNK)], sw[b]).wait()

        for b in range(NBUF - 1):
            start(b, b)

        def loop(p, carry):
            for b in range(NBUF):
                k = p * NBUF + b
                kn = k + NBUF - 1
                bn = (b + NBUF - 1) % NBUF

                @pl.when(kn < chunks_per_w)
                def _():
                    @pl.when(kn >= NBUF)
                    def _():
                        wait_writes(bn)
                    start(kn, bn)

                wait_gathers(k, b)
                for j in range(CHUNK // _NL):
                    idx = si_all[pl.ds(k * CHUNK + j * _NL, _NL)]
                    geb[b][pl.ds(j * _NL, _NL)] = plsc.load_gather(
                        batch_v, [idx])
                write(k, b)
            return carry

        lax.fori_loop(0, chunks_per_w // NBUF, loop, 0)
        for b in range(NBUF):
            wait_writes(b)

    return body


@functools.cache
def _sc_gather_kernel(n_edges):
    per_w = n_edges // _NW
    return functools.partial(
        pl.kernel,
        mesh=plsc.VectorSubcoreMesh(core_axis_name="c", subcore_axis_name="s"),
        compiler_params=pltpu.CompilerParams(needs_layout_passes=False),
        out_type=[
            jax.ShapeDtypeStruct((n_edges, D), jnp.float32),
            jax.ShapeDtypeStruct((n_edges, D), jnp.float32),
            jax.ShapeDtypeStruct((n_edges,), jnp.int32),
        ],
        scratch_types=(
            [pltpu.VMEM((N_NODES,), jnp.int32),
             pltpu.VMEM((per_w,), jnp.int32),
             pltpu.VMEM((per_w,), jnp.int32)]
            + [pltpu.VMEM((CHUNK, D), jnp.float32)] * (2 * NBUF)
            + [pltpu.VMEM((CHUNK,), jnp.int32)] * NBUF
            + [pltpu.SemaphoreType.DMA] * (2 * NBUF)
        ),
    )(_make_sc_gather_body(per_w))


def _tc_edge_body(gs_ref, gd_ref, ge_ref,
                  w1a_ref, w1b_ref, w2_ref, w3_ref,
                  b1_ref, b2_ref, b3_ref, wg_ref, acc_ref):
    i = pl.program_id(0)

    @pl.when(i == 0)
    def _init():
        acc_ref[...] = jnp.zeros_like(acc_ref)

    f32 = jnp.float32
    prec = jax.lax.Precision.HIGHEST
    h1 = jnp.maximum(
        jnp.dot(gs_ref[...], w1a_ref[...], preferred_element_type=f32,
                precision=prec)
        + jnp.dot(gd_ref[...], w1b_ref[...], preferred_element_type=f32,
                  precision=prec)
        + b1_ref[...], 0.0)
    h2 = jnp.maximum(
        jnp.dot(h1, w2_ref[...], preferred_element_type=f32, precision=prec)
        + b2_ref[...], 0.0)

    wgb = wg_ref[...][:, D:]
    vrow = jnp.sum(w3_ref[...] * wgb, axis=1)[None, :]
    c = jnp.sum(b3_ref[...] * wgb)
    s_col = jnp.sum(h2 * vrow, axis=1, keepdims=True) + c

    ge = ge_ref[0]
    iota_e = lax.broadcasted_iota(jnp.int32, (E_BLK, 128), 1)
    mask_e = iota_e == ge
    edge_part = jnp.sum(jnp.where(mask_e, s_col, 0.0), axis=0)

    acc_ref[0:1, :] = acc_ref[0:1, :] + edge_part[None, :]


@functools.cache
def _tc_edge_kernel(nblocks):
    return pl.pallas_call(
        _tc_edge_body,
        grid=(nblocks,),
        in_specs=[
            pl.BlockSpec((E_BLK, D), lambda i: (i, 0)),
            pl.BlockSpec((E_BLK, D), lambda i: (i, 0)),
            pl.BlockSpec((1, E_BLK, 1), lambda i: (i, 0, 0)),
            pl.BlockSpec((D, H), lambda i: (0, 0)),
            pl.BlockSpec((D, H), lambda i: (0, 0)),
            pl.BlockSpec((H, H), lambda i: (0, 0)),
            pl.BlockSpec((H, D), lambda i: (0, 0)),
            pl.BlockSpec((1, H), lambda i: (0, 0)),
            pl.BlockSpec((1, H), lambda i: (0, 0)),
            pl.BlockSpec((1, D), lambda i: (0, 0)),
            pl.BlockSpec((1, 2 * D), lambda i: (0, 0)),
        ],
        out_specs=pl.BlockSpec((8, 128), lambda i: (0, 0)),
        out_shape=jax.ShapeDtypeStruct((8, 128), jnp.float32),
    )


def _tc_node_body(x_ref, b_ref, wg_ref, acc_ref):
    i = pl.program_id(0)

    @pl.when(i == 0)
    def _init():
        acc_ref[...] = jnp.zeros_like(acc_ref)

    f32 = jnp.float32
    wga = wg_ref[...][:, :D]
    t_col = jnp.sum(x_ref[...] * wga, axis=1, keepdims=True)
    bt = b_ref[0]
    iota_n = lax.broadcasted_iota(jnp.int32, (N_BLK, 128), 1)
    mask_n = iota_n == bt
    node_part = jnp.sum(jnp.where(mask_n, t_col, 0.0), axis=0)
    cnt_part = jnp.sum(mask_n.astype(f32), axis=0)

    acc_ref[1:2, :] = acc_ref[1:2, :] + node_part[None, :]
    acc_ref[2:3, :] = acc_ref[2:3, :] + cnt_part[None, :]


_tc_node_kernel = pl.pallas_call(
    _tc_node_body,
    grid=(N_NODES // N_BLK,),
    in_specs=[
        pl.BlockSpec((N_BLK, D), lambda i: (i, 0)),
        pl.BlockSpec((1, N_BLK, 1), lambda i: (i, 0, 0)),
        pl.BlockSpec((1, 2 * D), lambda i: (0, 0)),
    ],
    out_specs=pl.BlockSpec((8, 128), lambda i: (0, 0)),
    out_shape=jax.ShapeDtypeStruct((8, 128), jnp.float32),
)


def kernel(x, edge_index, batch, W1, b1, W2, b2, W3, b3, Wg, bg):
    src = edge_index[0].astype(jnp.int32)
    dst = edge_index[1].astype(jnp.int32)
    srcp = jnp.pad(src, (0, E_PAD - N_EDGES))
    dstp = jnp.pad(dst, (0, E_PAD - N_EDGES))
    batch32 = batch.astype(jnp.int32)
    wg_row = Wg.reshape(1, 2 * D)

    weights = (W1[:D], W1[D:], W2, W3,
               b1.reshape(1, H), b2.reshape(1, H), b3.reshape(1, D), wg_row)

    accs = []
    for h in range(2):
        lo = h * E_HALF
        n_real = min(N_EDGES - lo, E_HALF)
        nblocks = n_real // E_BLK
        gs, gd, ge = _sc_gather_kernel(E_HALF)(
            x, lax.slice(srcp, (lo,), (lo + E_HALF,)),
            lax.slice(dstp, (lo,), (lo + E_HALF,)), batch32)
        ge3 = ge[:n_real].reshape(nblocks, E_BLK, 1)
        accs.append(_tc_edge_kernel(nblocks)(gs, gd, ge3, *weights))

    batch3 = batch32.reshape(N_NODES // N_BLK, N_BLK, 1)
    accs.append(_tc_node_kernel(x, batch3, wg_row))

    acc = accs[0] + accs[1] + accs[2]
    cnt = acc[2, :NUM_GRAPHS]
    sums = acc[0, :NUM_GRAPHS] + acc[1, :NUM_GRAPHS] + cnt * bg[0]
    return (sums / jnp.maximum(cnt, 1.0))[:, None]
